# Initial kernel scaffold; baseline (speedup 1.0000x reference)
import jax
import jax.numpy as jnp
from jax.experimental import pallas as pl


def kernel(high_items_pad, high_times_pad, high_len, user_id, items, ratings, times, seq_len, exposure_table, click_table, uid_table, rating_table, ts_w, ts_b, exp_w, exp_b, clk_w, clk_b):
    raise NotImplementedError("write your pallas kernel here")


# speedup vs baseline: 2.8848x; 2.8848x over previous
import functools

import jax
import jax.numpy as jnp
from jax import lax
from jax.experimental import pallas as pl
from jax.experimental.pallas import tpu as pltpu
from jax.experimental.pallas import tpu_sc as plsc

B = 1024
D = 128
SH = 200
SQ = 256
NSEQ = SH + SQ
NTOK = 520
FINAL = 514
NW = 32
RPW = B // NW


def _small_body(p1_ref, p2_ref, wc_ref, we_ref, b1_ref, b2_ref, oc_ref, oe_ref):
    oc_ref[...] = jnp.dot(p1_ref[...], wc_ref[...],
                          preferred_element_type=jnp.float32) + b1_ref[...]
    oe_ref[...] = jnp.dot(p2_ref[...], we_ref[...],
                          preferred_element_type=jnp.float32) + b2_ref[...]


def _small_call(p1, p2, wc, we, b1, b2):
    return pl.pallas_call(
        _small_body,
        out_shape=(jax.ShapeDtypeStruct((8, D), jnp.float32),
                   jax.ShapeDtypeStruct((8, D), jnp.float32)),
    )(p1, p2, wc, we, b1, b2)


_TBLK = 1024


def _transform_body(tab_ref, w_ref, c_ref, out_ref):
    out_ref[...] = jnp.dot(tab_ref[...], w_ref[...],
                           preferred_element_type=jnp.float32) + c_ref[...]


def _transform_call(table, w, cvec):
    n = table.shape[0]
    grid = (n + _TBLK - 1) // _TBLK
    return pl.pallas_call(
        _transform_body,
        grid=(grid,),
        in_specs=[
            pl.BlockSpec((_TBLK, D), lambda i: (i, 0)),
            pl.BlockSpec((D, D), lambda i: (0, 0)),
            pl.BlockSpec((1, D), lambda i: (0, 0)),
        ],
        out_specs=pl.BlockSpec((_TBLK, D), lambda i: (i, 0)),
        out_shape=jax.ShapeDtypeStruct((grid * _TBLK, D), jnp.float32),
    )(table, w, cvec)


_PBLK = 128


def _prep_body(ht_ref, tm_ref, hl_ref, sl_ref, t_ref, d_ref, len_ref):
    blk = pl.program_id(0)
    it = tm_ref[:, 256:257]
    t_c = jnp.log(it - ht_ref[...] + 1.0)
    t_e = jnp.log(it - tm_ref[:, :256] + 1.0)
    t_ref[...] = jnp.concatenate(
        [t_c, jnp.zeros((_PBLK, 8), jnp.float32), t_e,
         jnp.zeros((_PBLK, NTOK - NSEQ - 8), jnp.float32)], axis=1)

    hl = hl_ref[...]
    sl = sl_ref[...]
    z1 = 200 - hl
    z2 = 256 - jnp.maximum(sl - 1, 0)
    col = lax.broadcasted_iota(jnp.int32, (_PBLK, NTOK), 1)
    jl = jnp.where(col < z1, col, col + z2)
    ri = col - SH
    jr = jnp.where(ri < z2, z1 + ri, 201 + ri)
    j = jnp.where(col < SH, jl,
                  jnp.where(col < NSEQ, jr,
                            jnp.where(col == NSEQ, SH + z2,
                                      jnp.where((col >= 464) & (col < 519),
                                                col - 519, -1))))
    brow = blk * _PBLK + lax.broadcasted_iota(jnp.int32, (_PBLK, NTOK), 0)
    d_ref[...] = brow * FINAL + 55 + j
    len_ref[...] = hl + sl


def _prep_call(high_times_pad, times, high_len2, seq_len2):
    grid = B // _PBLK
    return pl.pallas_call(
        _prep_body,
        grid=(grid,),
        in_specs=[
            pl.BlockSpec((_PBLK, SH), lambda i: (i, 0)),
            pl.BlockSpec((_PBLK, 257), lambda i: (i, 0)),
            pl.BlockSpec((_PBLK, 1), lambda i: (i, 0)),
            pl.BlockSpec((_PBLK, 1), lambda i: (i, 0)),
        ],
        out_specs=(
            pl.BlockSpec((_PBLK, NTOK), lambda i: (i, 0)),
            pl.BlockSpec((_PBLK, NTOK), lambda i: (i, 0)),
            pl.BlockSpec((_PBLK, 1), lambda i: (i, 0)),
        ),
        out_shape=(
            jax.ShapeDtypeStruct((B, NTOK), jnp.float32),
            jax.ShapeDtypeStruct((B, NTOK), jnp.int32),
            jax.ShapeDtypeStruct((B, 1), jnp.int32),
        ),
    )(high_times_pad, times, high_len2, seq_len2)


_MESH = plsc.VectorSubcoreMesh(core_axis_name="c", subcore_axis_name="s")
_SCHUNK = 104


def _sc_body(click_hbm, expo_hbm, uid_hbm, raw_hbm, hitems_hbm, sitems_hbm,
             uids_hbm, tgts_hbm, tcat_hbm, dcat_hbm, srat_hbm,
             tvc_hbm, tve_hbm, rsep_hbm, out_hbm,
             ihi, isq, irat, tbuf, d0, d1, d2, d3, d4, rows, tvc, tve, rsep,
             uidv, tgtv, uidrows, tgtrows, duid, dtgt, sem, sem2):
    wid = lax.axis_index("s") * 2 + lax.axis_index("c")
    base = wid * RPW
    iota = lax.broadcasted_iota(jnp.int32, (16,), 0)
    zero16 = jnp.zeros((16,), jnp.float32)
    dbufs = [d0, d1, d2, d3, d4]

    pltpu.sync_copy(tvc_hbm, tvc)
    pltpu.sync_copy(tve_hbm, tve)
    pltpu.sync_copy(rsep_hbm, rsep)

    for k in range(8):
        rows[NSEQ, pl.ds(k * 16, 16)] = rsep[pl.ds(8 * D + k * 16, 16)]
    for r in range(NSEQ + 1, NTOK):
        for k in range(8):
            rows[r, pl.ds(k * 16, 16)] = zero16

    tvc_regs = [tvc[pl.ds(k * 16, 16)] for k in range(8)]
    tve_regs = [tve[pl.ds(k * 16, 16)] for k in range(8)]

    def row_step(r, _):
        b = base + r
        pltpu.sync_copy(hitems_hbm.at[pl.ds(b * SH, SH)], ihi)
        pltpu.sync_copy(sitems_hbm.at[pl.ds(b * SQ, SQ)], isq)
        pltpu.sync_copy(srat_hbm.at[pl.ds(b * SQ, SQ)], irat)
        pltpu.sync_copy(tcat_hbm.at[pl.ds(b * NTOK, NTOK)], tbuf)
        for c in range(5):
            pltpu.sync_copy(dcat_hbm.at[pl.ds(b * NTOK + c * _SCHUNK,
                                              _SCHUNK)], dbufs[c])
        cps = [
            pltpu.async_copy(click_hbm.at[ihi.at[pl.ds(0, 104)]],
                             rows.at[pl.ds(0, 104)], sem),
            pltpu.async_copy(click_hbm.at[ihi.at[pl.ds(104, 96)]],
                             rows.at[pl.ds(104, 96)], sem),
            pltpu.async_copy(expo_hbm.at[isq.at[pl.ds(0, 128)]],
                             rows.at[pl.ds(SH, 128)], sem2),
            pltpu.async_copy(expo_hbm.at[isq.at[pl.ds(128, 128)]],
                             rows.at[pl.ds(SH + 128, 128)], sem2),
        ]
        for cp in cps:
            cp.wait()

        def left_group(g, _):
            tg = tbuf[pl.ds(g * 16, 16)]
            for l in range(16):
                i = g * 16 + l
                t16 = jnp.broadcast_to(tg[l], (16,))
                for k in range(8):
                    sl = pl.ds(k * 16, 16)
                    rows[i, sl] = rows[i, sl] + t16 * tvc_regs[k]
            return 0

        lax.fori_loop(0, SH // 16, left_group, 0)
        tg = tbuf[pl.ds(192, 16)]
        for l in range(SH - 192):
            t16 = jnp.broadcast_to(tg[l], (16,))
            for k in range(8):
                sl = pl.ds(k * 16, 16)
                rows[192 + l, sl] = rows[192 + l, sl] + t16 * tvc_regs[k]

        def right_group(g, _):
            tg = tbuf[pl.ds(208 + g * 16, 16)]
            rg = irat[pl.ds(g * 16, 16)]
            for l in range(16):
                i = SH + g * 16 + l
                t16 = jnp.broadcast_to(tg[l], (16,))
                roff = rg[l] * D
                for k in range(8):
                    sl = pl.ds(k * 16, 16)
                    rv = rsep[pl.ds(roff + k * 16, 16)]
                    rows[i, sl] = rows[i, sl] + t16 * tve_regs[k] + rv
            return 0

        lax.fori_loop(0, SQ // 16, right_group, 0)

        wcps = [pltpu.async_copy(rows.at[pl.ds(c * _SCHUNK, _SCHUNK)],
                                 out_hbm.at[dbufs[c]], sem)
                for c in range(5)]
        for cp in wcps:
            cp.wait()
        return 0

    lax.fori_loop(0, RPW, row_step, 0)

    pltpu.sync_copy(uids_hbm.at[pl.ds(base, RPW)], uidv)
    pltpu.sync_copy(tgts_hbm.at[pl.ds(base, RPW)], tgtv)
    pltpu.async_copy(uid_hbm.at[uidv], uidrows, sem).wait()
    pltpu.async_copy(raw_hbm.at[tgtv], tgtrows, sem).wait()
    for k in range(RPW // 16):
        rowv = (base + k * 16 + iota) * FINAL
        duid[pl.ds(k * 16, 16)] = rowv + 512
        dtgt[pl.ds(k * 16, 16)] = rowv + 513
    pltpu.async_copy(uidrows, out_hbm.at[duid], sem).wait()
    pltpu.async_copy(tgtrows, out_hbm.at[dtgt], sem2).wait()


_sc_call = functools.partial(
    pl.kernel,
    out_type=jax.ShapeDtypeStruct((B * FINAL, D), jnp.float32),
    mesh=_MESH,
    scratch_types=[
        pltpu.VMEM((SH,), jnp.int32),
        pltpu.VMEM((SQ,), jnp.int32),
        pltpu.VMEM((SQ,), jnp.int32),
        pltpu.VMEM((NTOK,), jnp.float32),
        pltpu.VMEM((_SCHUNK,), jnp.int32),
        pltpu.VMEM((_SCHUNK,), jnp.int32),
        pltpu.VMEM((_SCHUNK,), jnp.int32),
        pltpu.VMEM((_SCHUNK,), jnp.int32),
        pltpu.VMEM((_SCHUNK,), jnp.int32),
        pltpu.VMEM((NTOK, D), jnp.float32),
        pltpu.VMEM((D,), jnp.float32),
        pltpu.VMEM((D,), jnp.float32),
        pltpu.VMEM((16 * D,), jnp.float32),
        pltpu.VMEM((RPW,), jnp.int32),
        pltpu.VMEM((RPW,), jnp.int32),
        pltpu.VMEM((RPW, D), jnp.float32),
        pltpu.VMEM((RPW, D), jnp.float32),
        pltpu.VMEM((RPW,), jnp.int32),
        pltpu.VMEM((RPW,), jnp.int32),
        pltpu.SemaphoreType.DMA,
        pltpu.SemaphoreType.DMA,
    ],
)(_sc_body)


def kernel(high_items_pad, high_times_pad, high_len, user_id, items, ratings,
           times, seq_len, exposure_table, click_table, uid_table,
           rating_table, ts_w, ts_b, exp_w, exp_b, clk_w, clk_b):
    i32 = jnp.int32
    f32 = jnp.float32
    seq_items = items[:, :SQ].astype(i32)
    item_id = items[:, SQ].astype(i32)
    seq_ratings = ratings[:, :SQ].astype(i32)
    item_rating = ratings[:, SQ]

    z = jnp.zeros((8, 3 * D), f32)
    p1 = z.at[0, D:2 * D].set(ts_w[0])
    p1 = p1.at[1, D:2 * D].set(ts_b).at[1, 2 * D:].set(rating_table[2])
    b1 = jnp.zeros((8, D), f32).at[1].set(clk_b)
    p2 = z.at[0, D:2 * D].set(ts_w[0])
    p2 = p2.at[1:7, 2 * D:].set(rating_table)
    p2 = p2.at[7, D:2 * D].set(ts_b)
    b2 = jnp.zeros((8, D), f32).at[7].set(exp_b)
    oc, oe = _small_call(p1, p2, clk_w, exp_w, b1, b2)

    click_tr = _transform_call(click_table, clk_w[:D], oc[1:2])
    expo_tr = _transform_call(exposure_table, exp_w[:D], oe[7:8])

    tcat, dcat, slen2 = _prep_call(
        high_times_pad, times, high_len[:, None].astype(i32),
        seq_len[:, None].astype(i32))

    rsep = jnp.zeros((16, D), f32)
    rsep = rsep.at[0:6].set(oe[1:7]).at[8].set(exposure_table[0])

    out_flat = _sc_call(
        click_tr, expo_tr, uid_table, exposure_table,
        high_items_pad.astype(i32).reshape(-1), seq_items.reshape(-1),
        user_id.astype(i32), item_id,
        tcat.reshape(-1), dcat.reshape(-1), seq_ratings.reshape(-1),
        oc[0], oe[0], rsep.reshape(-1))

    return (out_flat.reshape(B, FINAL, D), item_rating, slen2[:, 0], 2)


# speedup vs baseline: 3.2307x; 1.1199x over previous
import functools

import jax
import jax.numpy as jnp
from jax import lax
from jax.experimental import pallas as pl
from jax.experimental.pallas import tpu as pltpu
from jax.experimental.pallas import tpu_sc as plsc

B = 1024
D = 128
SH = 200
SQ = 256
NSEQ = SH + SQ
NTOK = 520
TW = 528
FINAL = 514
NW = 32
RPW = B // NW


def _small_body(p1_ref, p2_ref, wc_ref, we_ref, b1_ref, b2_ref, oc_ref, oe_ref):
    oc_ref[...] = jnp.dot(p1_ref[...], wc_ref[...],
                          preferred_element_type=jnp.float32) + b1_ref[...]
    oe_ref[...] = jnp.dot(p2_ref[...], we_ref[...],
                          preferred_element_type=jnp.float32) + b2_ref[...]


def _small_call(p1, p2, wc, we, b1, b2):
    return pl.pallas_call(
        _small_body,
        out_shape=(jax.ShapeDtypeStruct((8, D), jnp.float32),
                   jax.ShapeDtypeStruct((8, D), jnp.float32)),
    )(p1, p2, wc, we, b1, b2)


_TBLK = 1024


def _transform_body(tab_ref, w_ref, c_ref, out_ref):
    out_ref[...] = jnp.dot(tab_ref[...], w_ref[...],
                           preferred_element_type=jnp.float32) + c_ref[...]


def _transform_call(table, w, cvec):
    n = table.shape[0]
    grid = (n + _TBLK - 1) // _TBLK
    return pl.pallas_call(
        _transform_body,
        grid=(grid,),
        in_specs=[
            pl.BlockSpec((_TBLK, D), lambda i: (i, 0)),
            pl.BlockSpec((D, D), lambda i: (0, 0)),
            pl.BlockSpec((1, D), lambda i: (0, 0)),
        ],
        out_specs=pl.BlockSpec((_TBLK, D), lambda i: (i, 0)),
        out_shape=jax.ShapeDtypeStruct((grid * _TBLK, D), jnp.float32),
    )(table, w, cvec)


_PBLK = 128


def _prep_body(ht_ref, tm_ref, hl_ref, sl_ref, t_ref, d_ref, len_ref):
    blk = pl.program_id(0)
    it = tm_ref[:, 256:257]
    t_c = jnp.log(it - ht_ref[...] + 1.0)
    t_e = jnp.log(it - tm_ref[:, :256] + 1.0)
    t_ref[...] = jnp.concatenate(
        [t_c, jnp.zeros((_PBLK, 8), jnp.float32), t_e,
         jnp.zeros((_PBLK, TW - NSEQ - 8), jnp.float32)], axis=1)

    hl = hl_ref[...]
    sl = sl_ref[...]
    z1 = 200 - hl
    z2 = 256 - jnp.maximum(sl - 1, 0)
    col = lax.broadcasted_iota(jnp.int32, (_PBLK, NTOK), 1)
    jl = jnp.where(col < z1, col, col + z2)
    ri = col - SH
    jr = jnp.where(ri < z2, z1 + ri, 201 + ri)
    j = jnp.where(col < SH, jl,
                  jnp.where(col < NSEQ, jr,
                            jnp.where(col == NSEQ, SH + z2,
                                      jnp.where((col >= 464) & (col < 519),
                                                col - 519, -1))))
    brow = blk * _PBLK + lax.broadcasted_iota(jnp.int32, (_PBLK, NTOK), 0)
    d_ref[...] = brow * FINAL + 55 + j
    len_ref[...] = hl + sl


def _prep_call(high_times_pad, times, high_len2, seq_len2):
    grid = B // _PBLK
    return pl.pallas_call(
        _prep_body,
        grid=(grid,),
        in_specs=[
            pl.BlockSpec((_PBLK, SH), lambda i: (i, 0)),
            pl.BlockSpec((_PBLK, 257), lambda i: (i, 0)),
            pl.BlockSpec((_PBLK, 1), lambda i: (i, 0)),
            pl.BlockSpec((_PBLK, 1), lambda i: (i, 0)),
        ],
        out_specs=(
            pl.BlockSpec((_PBLK, TW), lambda i: (i, 0)),
            pl.BlockSpec((_PBLK, NTOK), lambda i: (i, 0)),
            pl.BlockSpec((_PBLK, 1), lambda i: (i, 0)),
        ),
        out_shape=(
            jax.ShapeDtypeStruct((B, TW), jnp.float32),
            jax.ShapeDtypeStruct((B, NTOK), jnp.int32),
            jax.ShapeDtypeStruct((B, 1), jnp.int32),
        ),
    )(high_times_pad, times, high_len2, seq_len2)


_MESH = plsc.VectorSubcoreMesh(core_axis_name="c", subcore_axis_name="s")
_SCHUNK = 104


def _sc_body(click_hbm, expo_hbm, uid_hbm, raw_hbm, hitems_hbm, sitems_hbm,
             uids_hbm, tgts_hbm, tcat_hbm, dcat_hbm, srat_hbm,
             tvc_hbm, tve_hbm, rsep_hbm, out_hbm,
             ihi_all, isq_all, irat_all, tcat_all,
             d0a, d1a, d2a, d3a, d4a, d0b, d1b, d2b, d3b, d4b,
             rows, tvc, tve, rsep,
             uidv, tgtv, uidrows, tgtrows, duid, dtgt,
             semp, semg, semg2, sems, semd):
    wid = lax.axis_index("s") * 2 + lax.axis_index("c")
    base = wid * RPW
    iota = lax.broadcasted_iota(jnp.int32, (16,), 0)
    zero16 = jnp.zeros((16,), jnp.float32)
    dsets = [[d0a, d1a, d2a, d3a, d4a], [d0b, d1b, d2b, d3b, d4b]]

    mcps = [
        pltpu.async_copy(hitems_hbm.at[pl.ds(base * SH, RPW * SH)], ihi_all,
                         semp),
        pltpu.async_copy(sitems_hbm.at[pl.ds(base * SQ, RPW * SQ)], isq_all,
                         semp),
        pltpu.async_copy(srat_hbm.at[pl.ds(base * SQ, RPW * SQ)], irat_all,
                         semp),
        pltpu.async_copy(tcat_hbm.at[pl.ds(base * TW, RPW * TW)], tcat_all,
                         semp),
    ]
    for c in range(5):
        pltpu.async_copy(dcat_hbm.at[pl.ds(base * NTOK + c * _SCHUNK,
                                           _SCHUNK)], dsets[0][c], semd)

    pltpu.sync_copy(tvc_hbm, tvc)
    pltpu.sync_copy(tve_hbm, tve)
    pltpu.sync_copy(rsep_hbm, rsep)

    pltpu.sync_copy(uids_hbm.at[pl.ds(base, RPW)], uidv)
    pltpu.sync_copy(tgts_hbm.at[pl.ds(base, RPW)], tgtv)
    pltpu.async_copy(uid_hbm.at[uidv], uidrows, semg).wait()
    pltpu.async_copy(raw_hbm.at[tgtv], tgtrows, semg).wait()
    for k in range(RPW // 16):
        rowv = (base + k * 16 + iota) * FINAL
        duid[pl.ds(k * 16, 16)] = rowv + 512
        dtgt[pl.ds(k * 16, 16)] = rowv + 513
    pltpu.async_copy(uidrows, out_hbm.at[duid], semg).wait()
    pltpu.async_copy(tgtrows, out_hbm.at[dtgt], semg2).wait()

    for k in range(8):
        rows[NSEQ, pl.ds(k * 16, 16)] = rsep[pl.ds(8 * D + k * 16, 16)]
    for r in range(NSEQ + 1, NTOK):
        for k in range(8):
            rows[r, pl.ds(k * 16, 16)] = zero16

    tvc_regs = [tvc[pl.ds(k * 16, 16)] for k in range(8)]
    tve_regs = [tve[pl.ds(k * 16, 16)] for k in range(8)]

    for cp in mcps:
        cp.wait()

    def pair_step(q, _):
        for half in range(2):
            r = 2 * q + half
            b = base + r
            dbufs = dsets[half]
            cps = [
                pltpu.async_copy(click_hbm.at[ihi_all.at[pl.ds(r * SH, 104)]],
                                 rows.at[pl.ds(0, 104)], semg),
                pltpu.async_copy(
                    click_hbm.at[ihi_all.at[pl.ds(r * SH + 104, 96)]],
                    rows.at[pl.ds(104, 96)], semg),
                pltpu.async_copy(expo_hbm.at[isq_all.at[pl.ds(r * SQ, 128)]],
                                 rows.at[pl.ds(SH, 128)], semg2),
                pltpu.async_copy(
                    expo_hbm.at[isq_all.at[pl.ds(r * SQ + 128, 128)]],
                    rows.at[pl.ds(SH + 128, 128)], semg2),
            ]
            cps[0].wait()
            cps[1].wait()

            def left_group(g, _):
                tg = tcat_all[pl.ds(r * TW + g * 16, 16)]
                for l in range(16):
                    i = g * 16 + l
                    t16 = jnp.broadcast_to(tg[l], (16,))
                    for k in range(8):
                        sl = pl.ds(k * 16, 16)
                        rows[i, sl] = rows[i, sl] + t16 * tvc_regs[k]
                return 0

            lax.fori_loop(0, SH // 16, left_group, 0)
            tg = tcat_all[pl.ds(r * TW + 192, 16)]
            for l in range(SH - 192):
                t16 = jnp.broadcast_to(tg[l], (16,))
                for k in range(8):
                    sl = pl.ds(k * 16, 16)
                    rows[192 + l, sl] = rows[192 + l, sl] + t16 * tvc_regs[k]

            cps[2].wait()
            cps[3].wait()

            def right_group(g, _):
                tg = tcat_all[pl.ds(r * TW + 208 + g * 16, 16)]
                rg = irat_all[pl.ds(r * SQ + g * 16, 16)]
                for l in range(16):
                    i = SH + g * 16 + l
                    t16 = jnp.broadcast_to(tg[l], (16,))
                    roff = rg[l] * D
                    for k in range(8):
                        sl = pl.ds(k * 16, 16)
                        rv = rsep[pl.ds(roff + k * 16, 16)]
                        rows[i, sl] = rows[i, sl] + t16 * tve_regs[k] + rv
                return 0

            lax.fori_loop(0, SQ // 16, right_group, 0)

            for c in range(5):
                pltpu.make_async_copy(
                    dcat_hbm.at[pl.ds(b * NTOK + c * _SCHUNK, _SCHUNK)],
                    dbufs[c], semd).wait()
            wcps = [pltpu.async_copy(rows.at[pl.ds(c * _SCHUNK, _SCHUNK)],
                                     out_hbm.at[dbufs[c]], sems)
                    for c in range(5)]
            for cp in wcps:
                cp.wait()

            @pl.when(r + 1 < RPW)
            def _():
                nb = b + 1
                for c in range(5):
                    pltpu.async_copy(
                        dcat_hbm.at[pl.ds(nb * NTOK + c * _SCHUNK, _SCHUNK)],
                        dsets[1 - half][c], semd)
        return 0

    lax.fori_loop(0, RPW // 2, pair_step, 0)


_sc_call = functools.partial(
    pl.kernel,
    out_type=jax.ShapeDtypeStruct((B * FINAL, D), jnp.float32),
    mesh=_MESH,
    scratch_types=(
        [
            pltpu.VMEM((RPW * SH,), jnp.int32),
            pltpu.VMEM((RPW * SQ,), jnp.int32),
            pltpu.VMEM((RPW * SQ,), jnp.int32),
            pltpu.VMEM((RPW * TW,), jnp.float32),
        ]
        + [pltpu.VMEM((_SCHUNK,), jnp.int32) for _ in range(10)]
        + [
            pltpu.VMEM((NTOK, D), jnp.float32),
            pltpu.VMEM((D,), jnp.float32),
            pltpu.VMEM((D,), jnp.float32),
            pltpu.VMEM((16 * D,), jnp.float32),
            pltpu.VMEM((RPW,), jnp.int32),
            pltpu.VMEM((RPW,), jnp.int32),
            pltpu.VMEM((RPW, D), jnp.float32),
            pltpu.VMEM((RPW, D), jnp.float32),
            pltpu.VMEM((RPW,), jnp.int32),
            pltpu.VMEM((RPW,), jnp.int32),
            pltpu.SemaphoreType.DMA,
            pltpu.SemaphoreType.DMA,
            pltpu.SemaphoreType.DMA,
            pltpu.SemaphoreType.DMA,
            pltpu.SemaphoreType.DMA,
        ]
    ),
)(_sc_body)


def kernel(high_items_pad, high_times_pad, high_len, user_id, items, ratings,
           times, seq_len, exposure_table, click_table, uid_table,
           rating_table, ts_w, ts_b, exp_w, exp_b, clk_w, clk_b):
    i32 = jnp.int32
    f32 = jnp.float32
    seq_items = items[:, :SQ].astype(i32)
    item_id = items[:, SQ].astype(i32)
    seq_ratings = ratings[:, :SQ].astype(i32)
    item_rating = ratings[:, SQ]

    z = jnp.zeros((8, 3 * D), f32)
    p1 = z.at[0, D:2 * D].set(ts_w[0])
    p1 = p1.at[1, D:2 * D].set(ts_b).at[1, 2 * D:].set(rating_table[2])
    b1 = jnp.zeros((8, D), f32).at[1].set(clk_b)
    p2 = z.at[0, D:2 * D].set(ts_w[0])
    p2 = p2.at[1:7, 2 * D:].set(rating_table)
    p2 = p2.at[7, D:2 * D].set(ts_b)
    b2 = jnp.zeros((8, D), f32).at[7].set(exp_b)
    oc, oe = _small_call(p1, p2, clk_w, exp_w, b1, b2)

    click_tr = _transform_call(click_table, clk_w[:D], oc[1:2])
    expo_tr = _transform_call(exposure_table, exp_w[:D], oe[7:8])

    tcat, dcat, slen2 = _prep_call(
        high_times_pad, times, high_len[:, None].astype(i32),
        seq_len[:, None].astype(i32))

    rsep = jnp.zeros((16, D), f32)
    rsep = rsep.at[0:6].set(oe[1:7]).at[8].set(exposure_table[0])

    out_flat = _sc_call(
        click_tr, expo_tr, uid_table, exposure_table,
        high_items_pad.astype(i32).reshape(-1), seq_items.reshape(-1),
        user_id.astype(i32), item_id,
        tcat.reshape(-1), dcat.reshape(-1), seq_ratings.reshape(-1),
        oc[0], oe[0], rsep.reshape(-1))

    return (out_flat.reshape(B, FINAL, D), item_rating, slen2[:, 0], 2)
